# Initial kernel scaffold; baseline (speedup 1.0000x reference)
import jax
import jax.numpy as jnp
from jax.experimental import pallas as pl


def kernel(inputs, codebooks):
    raise NotImplementedError("write your pallas kernel here")


# speedup vs baseline: 1.0374x; 1.0374x over previous
import functools

import jax
import jax.numpy as jnp
from jax.experimental import pallas as pl

_NUM_STAGES = 4
_K = 1024
_D = 256
_BLK = 1024


def _rvq_kernel(x_ref, cb_ref, quant_ref, codes_ref, loss_ref):
    i = pl.program_id(0)
    x = x_ref[...]
    r = x
    qsum = jnp.zeros_like(x)
    loss = jnp.float32(0.0)
    for s in range(_NUM_STAGES):
        cb = cb_ref[s]
        a = jnp.sum(r * r, axis=1, keepdims=True)
        b = jax.lax.dot_general(
            r, cb, (((1,), (1,)), ((), ())),
            preferred_element_type=jnp.float32)
        c = jnp.sum(cb * cb, axis=1)[None, :]
        dists = a - 2.0 * b + c
        m = jnp.min(dists, axis=1, keepdims=True)
        iota = jax.lax.broadcasted_iota(jnp.int32, dists.shape, 1)
        idx = jnp.min(jnp.where(dists == m, iota, _K), axis=1)
        onehot = (iota == idx[:, None]).astype(jnp.float32)
        q = jax.lax.dot_general(
            onehot, cb, (((1,), (0,)), ((), ())),
            preferred_element_type=jnp.float32,
            precision=jax.lax.Precision.HIGHEST)
        loss = loss + jnp.sum((q - r) * (q - r))
        codes_ref[s, :] = idx
        qsum = qsum + q
        r = r - q
    quant_ref[...] = x + (qsum - x)

    loss2d = loss.reshape(1, 1)

    @pl.when(i == 0)
    def _init():
        loss_ref[...] = loss2d

    @pl.when(i != 0)
    def _acc():
        loss_ref[...] += loss2d


@functools.partial(jax.jit, static_argnames=())
def kernel(inputs, codebooks):
    B, N, D = inputs.shape
    tokens = B * N
    flat = inputs.reshape(tokens, D)
    grid = tokens // _BLK
    quant, codes, loss = pl.pallas_call(
        _rvq_kernel,
        grid=(grid,),
        in_specs=[
            pl.BlockSpec((_BLK, D), lambda i: (i, 0)),
            pl.BlockSpec((_NUM_STAGES, _K, D), lambda i: (0, 0, 0)),
        ],
        out_specs=[
            pl.BlockSpec((_BLK, D), lambda i: (i, 0)),
            pl.BlockSpec((_NUM_STAGES, _BLK), lambda i: (0, i)),
            pl.BlockSpec((1, 1), lambda i: (0, 0)),
        ],
        out_shape=[
            jax.ShapeDtypeStruct((tokens, D), jnp.float32),
            jax.ShapeDtypeStruct((_NUM_STAGES, tokens), jnp.int32),
            jax.ShapeDtypeStruct((1, 1), jnp.float32),
        ],
    )(flat, codebooks)
    scale = (1.0 + 0.25) / jnp.float32(tokens * D)
    total_loss = loss[0, 0] * scale
    quantized = quant.reshape(B, N, D)
    codes = codes.reshape(_NUM_STAGES, B, N)
    return quantized, total_loss, codes


# speedup vs baseline: 1.7326x; 1.6702x over previous
import functools

import jax
import jax.numpy as jnp
from jax.experimental import pallas as pl

_NUM_STAGES = 4
_K = 1024
_D = 256
_BLK = 1024


def _rvq_kernel(x_ref, cb_ref, quant_ref, codes_ref, loss_ref):
    i = pl.program_id(0)
    x = x_ref[...]
    r = x
    qsum = jnp.zeros_like(x)
    loss = jnp.float32(0.0)
    iota_f = jax.lax.broadcasted_iota(
        jnp.int32, (x.shape[0], _K), 1).astype(jnp.float32)
    for s in range(_NUM_STAGES):
        cb = cb_ref[s]
        a = jnp.sum(r * r, axis=1, keepdims=True)
        b = jax.lax.dot_general(
            r, cb, (((1,), (1,)), ((), ())),
            preferred_element_type=jnp.float32)
        c = jnp.sum(cb * cb, axis=1)[None, :]
        dists = a - 2.0 * b + c
        m = jnp.min(dists, axis=1, keepdims=True)
        masked = jnp.where(dists == m, iota_f, jnp.float32(_K))
        idxf = jnp.min(masked, axis=1, keepdims=True)
        onehot = (masked == idxf).astype(jnp.bfloat16)
        cb_hi = cb.astype(jnp.bfloat16)
        r1 = cb - cb_hi.astype(jnp.float32)
        cb_mid = r1.astype(jnp.bfloat16)
        cb_lo = (r1 - cb_mid.astype(jnp.float32)).astype(jnp.bfloat16)

        def _oh_dot(mat):
            return jax.lax.dot_general(
                onehot, mat, (((1,), (0,)), ((), ())),
                preferred_element_type=jnp.float32)

        q = (_oh_dot(cb_hi) + _oh_dot(cb_mid)) + _oh_dot(cb_lo)
        loss = loss + jnp.sum((q - r) * (q - r))
        codes_ref[s, :] = idxf[:, 0].astype(jnp.int32)
        qsum = qsum + q
        r = r - q
    quant_ref[...] = x + (qsum - x)

    loss2d = loss.reshape(1, 1)

    @pl.when(i == 0)
    def _init():
        loss_ref[...] = loss2d

    @pl.when(i != 0)
    def _acc():
        loss_ref[...] += loss2d


@functools.partial(jax.jit, static_argnames=())
def kernel(inputs, codebooks):
    B, N, D = inputs.shape
    tokens = B * N
    flat = inputs.reshape(tokens, D)
    grid = tokens // _BLK
    quant, codes, loss = pl.pallas_call(
        _rvq_kernel,
        grid=(grid,),
        in_specs=[
            pl.BlockSpec((_BLK, D), lambda i: (i, 0)),
            pl.BlockSpec((_NUM_STAGES, _K, D), lambda i: (0, 0, 0)),
        ],
        out_specs=[
            pl.BlockSpec((_BLK, D), lambda i: (i, 0)),
            pl.BlockSpec((_NUM_STAGES, _BLK), lambda i: (0, i)),
            pl.BlockSpec((1, 1), lambda i: (0, 0)),
        ],
        out_shape=[
            jax.ShapeDtypeStruct((tokens, D), jnp.float32),
            jax.ShapeDtypeStruct((_NUM_STAGES, tokens), jnp.int32),
            jax.ShapeDtypeStruct((1, 1), jnp.float32),
        ],
    )(flat, codebooks)
    scale = (1.0 + 0.25) / jnp.float32(tokens * D)
    total_loss = loss[0, 0] * scale
    quantized = quant.reshape(B, N, D)
    codes = codes.reshape(_NUM_STAGES, B, N)
    return quantized, total_loss, codes
